# baseline (speedup 1.0000x reference)
import functools

import jax
import jax.numpy as jnp
from jax import lax
from jax.experimental import pallas as pl
from jax.experimental.pallas import tpu as pltpu
from jax.experimental.pallas import tpu_sc as plsc

N = 10000
H = 128
E = 320000
BN_EPS = 1e-5

NC = 2
NS = 16
NW = NC * NS
NP = 10240
CHUNK = 128
EP = 327680
ROWS_PER_TILE = NP // NS
CHUNKS_PER_WORKER = EP // CHUNK // NW


def _sc_agg(xp, srcm, dstm, zeros):
    mesh = plsc.VectorSubcoreMesh(core_axis_name="c", subcore_axis_name="s")

    def body(xh, sh, dh, zh, out, aggs, sidx0, didx0, sidx1, didx1,
             gbufs, sem_g, sem_s, sem_i):
        c = lax.axis_index("c")
        s = lax.axis_index("s")
        r0 = s * ROWS_PER_TILE
        sidx = (sidx0, sidx1)
        didx = (didx0, didx1)
        pltpu.sync_copy(zh.at[pl.ds(r0, ROWS_PER_TILE)],
                        aggs.at[pl.ds(r0, ROWS_PER_TILE)])
        plsc.subcore_barrier()

        c0 = s * (CHUNKS_PER_WORKER * NC)
        T = CHUNKS_PER_WORKER * NC

        def idx_start(j, b):
            pltpu.async_copy(sh.at[c0 + j], sidx[b], sem_i.at[b])
            pltpu.async_copy(dh.at[c0 + j], didx[b], sem_i.at[b])

        def idx_wait(b):
            pltpu.make_async_copy(sh.at[c0], sidx[b], sem_i.at[b]).wait()
            pltpu.make_async_copy(dh.at[c0], didx[b], sem_i.at[b]).wait()

        def gather_start(b):
            pltpu.async_copy(xh.at[sidx[b]], gbufs.at[b], sem_g.at[b])

        def gather_wait(b):
            pltpu.make_async_copy(xh.at[sidx[b]], gbufs.at[b],
                                  sem_g.at[b]).wait()

        def scatter_start(b):
            pltpu.async_copy(gbufs.at[b], aggs.at[didx[b]], sem_s.at[b],
                             add=True)

        def scatter_wait(b):
            pltpu.make_async_copy(gbufs.at[b], aggs.at[didx[b]],
                                  sem_s.at[b]).wait()

        @pl.when(c == 0)
        def _pipeline():
            idx_start(0, 0)
            idx_wait(0)
            gather_start(0)
            idx_start(1, 1)

            def group(g, carry):
                gather_wait(0)

                @pl.when(g > 0)
                def _():
                    scatter_wait(1)
                idx_wait(1)
                gather_start(1)

                @pl.when(g + 2 < T)
                def _():
                    idx_start(g + 2, 0)
                scatter_start(0)

                gather_wait(1)
                scatter_wait(0)

                @pl.when(g + 2 < T)
                def _():
                    idx_wait(0)
                    gather_start(0)

                    @pl.when(g + 3 < T)
                    def _():
                        idx_start(g + 3, 1)
                scatter_start(1)
                return carry

            lax.fori_loop(0, T // 2, lambda i, cr: group(i * 2, cr), 0)
            scatter_wait(1)

        plsc.subcore_barrier()
        pltpu.sync_copy(aggs.at[pl.ds(r0, ROWS_PER_TILE)],
                        out.at[c, pl.ds(r0, ROWS_PER_TILE)])

    kfn = pl.kernel(
        body,
        out_type=jax.ShapeDtypeStruct((NC, NP, H), jnp.float32),
        mesh=mesh,
        scratch_types=[
            pltpu.VMEM_SHARED((NP, H), jnp.float32),
            pltpu.VMEM((CHUNK,), jnp.int32),
            pltpu.VMEM((CHUNK,), jnp.int32),
            pltpu.VMEM((CHUNK,), jnp.int32),
            pltpu.VMEM((CHUNK,), jnp.int32),
            pltpu.VMEM((2, CHUNK, H), jnp.float32),
            pltpu.SemaphoreType.DMA((2,)),
            pltpu.SemaphoreType.DMA((2,)),
            pltpu.SemaphoreType.DMA((2,)),
        ],
    )
    return kfn(xp, srcm, dstm, zeros)


def _mlp_body(mode, h_ref, a_ref, w1, b1, g, be, w2, b2, prev, o_ref):
    agg = a_ref[0, :N, :] + a_ref[1, :N, :]
    h = h_ref[...] + agg
    y = jnp.dot(h, w1[...], preferred_element_type=jnp.float32) + b1[...]
    mu = jnp.mean(y, axis=0, keepdims=True)
    var = jnp.mean((y - mu) * (y - mu), axis=0, keepdims=True)
    t = (y - mu) * lax.rsqrt(var + BN_EPS) * g[...] + be[...]
    t = jnp.maximum(t, 0.0)
    z = jnp.dot(t, w2[...], preferred_element_type=jnp.float32) + b2[...]
    if mode == 0:
        o_ref[...] = jnp.where(z >= 0, z, 0.01 * z)
    else:
        o_ref[...] = prev[...] + z


def _mlp_tc(mode, hin, agg, W1, b1, g, be, W2, b2, prev):
    return pl.pallas_call(
        functools.partial(_mlp_body, mode),
        out_shape=jax.ShapeDtypeStruct((N, H), jnp.float32),
    )(hin, agg, W1, b1.reshape(1, H), g.reshape(1, H), be.reshape(1, H),
      W2, b2.reshape(1, H), prev)


def _pad(h):
    return jnp.concatenate([h, jnp.zeros((NP - N, H), h.dtype)], axis=0)


def kernel(x, edge_index, W1a, b1a, g1a, be1a, W2a, b2a,
           W1b, b1b, g1b, be1b, W2b, b2b):
    ei = edge_index.astype(jnp.int32)
    pad = jnp.full((EP - E,), N, jnp.int32)
    srcm = jnp.concatenate([ei[0], pad]).reshape(EP // CHUNK, CHUNK)
    dstm = jnp.concatenate([ei[1], pad]).reshape(EP // CHUNK, CHUNK)
    zeros = jnp.zeros((NP, H), jnp.float32)

    agg1 = _sc_agg(_pad(x), srcm, dstm, zeros)
    h1 = _mlp_tc(0, x, agg1, W1a, b1a, g1a, be1a, W2a, b2a, x)
    agg2 = _sc_agg(_pad(h1), srcm, dstm, zeros)
    out = _mlp_tc(1, h1, agg2, W1b, b1b, g1b, be1b, W2b, b2b, h1)
    return out


# speedup vs baseline: 1.6969x; 1.6969x over previous
import functools

import jax
import jax.numpy as jnp
from jax import lax
from jax.experimental import pallas as pl
from jax.experimental.pallas import tpu as pltpu
from jax.experimental.pallas import tpu_sc as plsc

N = 10000
H = 128
E = 320000
BN_EPS = 1e-5

NC = 2
NS = 16
NW = NC * NS
NP = 10048
CHUNK = 128
EP = 327680
HW = H // 2
ROWS_PER_TILE = NP // NS
CHUNKS_PER_WORKER = EP // CHUNK // NW
L = 16


def _sc_agg(xpk, srcm, dstm, zeros):
    mesh = plsc.VectorSubcoreMesh(core_axis_name="c", subcore_axis_name="s")

    def body(xk, sh, dh, zh, out, aggs, sidx0, didx0, sidx1, didx1,
             gbuf0, gbuf1, fbuf0, fbuf1, sem_g, sem_s, sem_i):
        gbufs = (gbuf0, gbuf1)
        fbufs = (fbuf0, fbuf1)
        c = lax.axis_index("c")
        s = lax.axis_index("s")
        r0 = s * ROWS_PER_TILE
        sidx = (sidx0, sidx1)
        didx = (didx0, didx1)
        pltpu.sync_copy(zh.at[pl.ds(r0, ROWS_PER_TILE)],
                        aggs.at[pl.ds(r0, ROWS_PER_TILE)])
        plsc.subcore_barrier()

        c0 = (s * NC + c) * CHUNKS_PER_WORKER
        T = CHUNKS_PER_WORKER

        def idx_start(j, b):
            pltpu.async_copy(sh.at[c0 + j], sidx[b], sem_i.at[b])
            pltpu.async_copy(dh.at[c0 + j], didx[b], sem_i.at[b])

        def idx_wait(b):
            pltpu.make_async_copy(sh.at[c0], sidx[b], sem_i.at[b]).wait()
            pltpu.make_async_copy(dh.at[c0], didx[b], sem_i.at[b]).wait()

        def gather_start(b):
            pltpu.async_copy(xk.at[sidx[b]], gbufs[b], sem_g.at[b])

        def gather_wait(b):
            pltpu.make_async_copy(xk.at[sidx[b]], gbufs[b],
                                  sem_g.at[b]).wait()

        def scatter_start(b):
            pltpu.async_copy(fbufs[b], aggs.at[didx[b]], sem_s.at[b],
                             add=True)

        def scatter_wait(b):
            pltpu.make_async_copy(fbufs[b], aggs.at[didx[b]],
                                  sem_s.at[b]).wait()

        def widen(b):
            gb = gbufs[b]
            fb = fbufs[b]
            hi_mask = jnp.int32(-65536)

            def row(r, cr):
                for cg in range(HW // L):
                    v = gb[r, pl.ds(cg * L, L)]
                    fb[r, pl.ds(cg * L, L)] = plsc.bitcast(
                        lax.shift_left(v, 16), jnp.float32)
                    fb[r, pl.ds(HW + cg * L, L)] = plsc.bitcast(
                        lax.bitwise_and(v, hi_mask), jnp.float32)
                return cr

            lax.fori_loop(0, CHUNK, row, 0, unroll=4)

        idx_start(0, 0)
        idx_wait(0)
        gather_start(0)
        idx_start(1, 1)

        def group(g, carry):
            gather_wait(0)
            idx_wait(1)
            gather_start(1)
            widen(0)

            @pl.when(g > 0)
            def _():
                scatter_wait(1)

            @pl.when(g + 2 < T)
            def _():
                idx_start(g + 2, 0)
            scatter_start(0)

            gather_wait(1)

            @pl.when(g + 2 < T)
            def _():
                idx_wait(0)
                gather_start(0)
            widen(1)
            scatter_wait(0)

            @pl.when(g + 3 < T)
            def _():
                idx_start(g + 3, 1)
            scatter_start(1)
            return carry

        lax.fori_loop(0, T // 2, lambda i, cr: group(i * 2, cr), 0)
        scatter_wait(1)
        plsc.subcore_barrier()
        pltpu.sync_copy(aggs.at[pl.ds(r0, ROWS_PER_TILE)],
                        out.at[c, pl.ds(r0, ROWS_PER_TILE)])

    kfn = pl.kernel(
        body,
        out_type=jax.ShapeDtypeStruct((NC, NP, H), jnp.float32),
        mesh=mesh,
        compiler_params=pltpu.CompilerParams(use_tc_tiling_on_sc=False,
                                             needs_layout_passes=False),
        scratch_types=[
            pltpu.VMEM_SHARED((NP, H), jnp.float32),
            pltpu.VMEM((CHUNK,), jnp.int32),
            pltpu.VMEM((CHUNK,), jnp.int32),
            pltpu.VMEM((CHUNK,), jnp.int32),
            pltpu.VMEM((CHUNK,), jnp.int32),
            pltpu.VMEM((CHUNK, HW), jnp.int32),
            pltpu.VMEM((CHUNK, HW), jnp.int32),
            pltpu.VMEM((CHUNK, H), jnp.float32),
            pltpu.VMEM((CHUNK, H), jnp.float32),
            pltpu.SemaphoreType.DMA((2,)),
            pltpu.SemaphoreType.DMA((2,)),
            pltpu.SemaphoreType.DMA((2,)),
        ],
    )
    return kfn(xpk, srcm, dstm, zeros)


def _mlp_body(mode, h_ref, a_ref, w1, b1, g, be, w2, b2, prev, o_ref):
    agg = a_ref[0, :N, :] + a_ref[1, :N, :]
    h = h_ref[...] + agg
    y = jnp.dot(h, w1[...], preferred_element_type=jnp.float32) + b1[...]
    mu = jnp.mean(y, axis=0, keepdims=True)
    var = jnp.mean((y - mu) * (y - mu), axis=0, keepdims=True)
    t = (y - mu) * lax.rsqrt(var + BN_EPS) * g[...] + be[...]
    t = jnp.maximum(t, 0.0)
    z = jnp.dot(t, w2[...], preferred_element_type=jnp.float32) + b2[...]
    if mode == 0:
        o_ref[...] = jnp.where(z >= 0, z, 0.01 * z)
    else:
        o_ref[...] = prev[...] + z


def _mlp_tc(mode, hin, agg, W1, b1, g, be, W2, b2, prev):
    return pl.pallas_call(
        functools.partial(_mlp_body, mode),
        out_shape=jax.ShapeDtypeStruct((N, H), jnp.float32),
    )(hin, agg, W1, b1.reshape(1, H), g.reshape(1, H), be.reshape(1, H),
      W2, b2.reshape(1, H), prev)


def _pack(h):
    hp = jnp.concatenate([h, jnp.zeros((NP - N, H), h.dtype)], axis=0)
    hb = hp.astype(jnp.bfloat16)
    pairs = jnp.stack([hb[:, :HW], hb[:, HW:]], axis=-1)
    return lax.bitcast_convert_type(pairs, jnp.int32)


def kernel(x, edge_index, W1a, b1a, g1a, be1a, W2a, b2a,
           W1b, b1b, g1b, be1b, W2b, b2b):
    ei = edge_index.astype(jnp.int32)
    pad = jnp.full((EP - E,), N, jnp.int32)
    srcm = jnp.concatenate([ei[0], pad]).reshape(EP // CHUNK, CHUNK)
    dstm = jnp.concatenate([ei[1], pad]).reshape(EP // CHUNK, CHUNK)
    zeros = jnp.zeros((NP, H), jnp.float32)

    agg1 = _sc_agg(_pack(x), srcm, dstm, zeros)
    h1 = _mlp_tc(0, x, agg1, W1a, b1a, g1a, be1a, W2a, b2a, x)
    agg2 = _sc_agg(_pack(h1), srcm, dstm, zeros)
    out = _mlp_tc(1, h1, agg2, W1b, b1b, g1b, be1b, W2b, b2b, h1)
    return out
